# Initial kernel scaffold; baseline (speedup 1.0000x reference)
import functools

import jax
import jax.numpy as jnp
from jax import lax
from jax.experimental import pallas as pl
from jax.experimental.pallas import tpu as pltpu
from jax.experimental.pallas import tpu_sc as plsc

N = 10000
D = 128
E = 100000
B = 8192

NP = 10240
DUMP = N
NC, NS, NW = 2, 16, 32
K = 128
ROWS_W = 25
EPAD = NW * ROWS_W * K
ROWS_REL = NW * ROWS_W
DEG_ROWS_W = 6 * ROWS_W
RPS = NP // NS
DRPS = 6 * NP // NS

_mesh = plsc.VectorSubcoreMesh(core_axis_name="c", subcore_axis_name="s",
                               num_cores=NC, num_subcores=NS)


def _zfill_f32(ref, nrows, ncols16):
    @pl.loop(0, nrows)
    def _(i):
        for c in range(ncols16):
            ref[i, pl.ds(c * 16, 16)] = jnp.zeros((16,), jnp.float32)


def _deg_kernel_body(didx_hbm, out_hbm, acc, ibuf, ones_v, zv):
    cid = lax.axis_index("c")
    sid = lax.axis_index("s")
    wid = cid * NS + sid

    @pl.loop(0, K)
    def _(i):
        ones_v[i, :] = jnp.ones((16,), jnp.float32)
    _zfill_f32(zv, K, 1)

    @pl.loop(0, DRPS // K)
    def _(t):
        pltpu.sync_copy(zv, acc.at[pl.ds(sid * DRPS + t * K, K)])
    plsc.subcore_barrier()

    pltpu.sync_copy(didx_hbm.at[pl.ds(wid * DEG_ROWS_W, DEG_ROWS_W)], ibuf)

    @pl.loop(0, DEG_ROWS_W)
    def _(j):
        pltpu.sync_copy(ones_v, acc.at[ibuf.at[j]], add=True)
    plsc.subcore_barrier()

    pltpu.sync_copy(acc.at[pl.ds(sid * DRPS, DRPS)],
                    out_hbm.at[pl.ds(cid * 6 * NP + sid * DRPS, DRPS)])


def _sc_degrees(deg_idx):
    f = pl.kernel(
        _deg_kernel_body,
        out_type=jax.ShapeDtypeStruct((NC * 6 * NP, 16), jnp.float32),
        mesh=_mesh,
        scratch_types=[
            pltpu.VMEM_SHARED((6 * NP, 16), jnp.float32),
            pltpu.VMEM((DEG_ROWS_W, K), jnp.int32),
            pltpu.VMEM((K, 16), jnp.float32),
            pltpu.VMEM((K, 16), jnp.float32),
        ],
    )
    return f(deg_idx)


def _edge_kernel_body(tab_hbm, sidx_hbm, didx_hbm, out_hbm,
                      acc, sbuf, dbuf, rows, zv):
    cid = lax.axis_index("c")
    sid = lax.axis_index("s")
    wid = cid * NS + sid

    _zfill_f32(zv, K, D // 16)

    for r in range(3):
        @pl.loop(0, RPS // K)
        def _(t):
            pltpu.sync_copy(zv, acc.at[pl.ds(sid * RPS + t * K, K)])
        plsc.subcore_barrier()

        base = r * ROWS_REL + wid * ROWS_W
        pltpu.sync_copy(sidx_hbm.at[pl.ds(base, ROWS_W)], sbuf)
        pltpu.sync_copy(didx_hbm.at[pl.ds(base, ROWS_W)], dbuf)

        @pl.loop(0, ROWS_W)
        def _(j):
            pltpu.sync_copy(tab_hbm.at[sbuf.at[j]], rows)
            pltpu.sync_copy(rows, acc.at[dbuf.at[j]], add=True)
        plsc.subcore_barrier()

        pltpu.sync_copy(acc.at[pl.ds(sid * RPS, RPS)],
                        out_hbm.at[pl.ds((cid * 3 + r) * NP + sid * RPS, RPS)])
        plsc.subcore_barrier()


def _sc_edge_pass(tab3, sidx, didx):
    f = pl.kernel(
        _edge_kernel_body,
        out_type=jax.ShapeDtypeStruct((NC * 3 * NP, D), jnp.float32),
        mesh=_mesh,
        scratch_types=[
            pltpu.VMEM_SHARED((NP, D), jnp.float32),
            pltpu.VMEM((ROWS_W, K), jnp.int32),
            pltpu.VMEM((ROWS_W, K), jnp.int32),
            pltpu.VMEM((K, D), jnp.float32),
            pltpu.VMEM((K, D), jnp.float32),
        ],
    )
    return f(tab3, sidx, didx)


_GROWS_W = (2 * B) // K // NW


def _final_gather_body(h2_hbm, idx_hbm, out_hbm, ibuf, rows):
    wid = lax.axis_index("c") * NS + lax.axis_index("s")
    pltpu.sync_copy(idx_hbm.at[pl.ds(wid * _GROWS_W, _GROWS_W)], ibuf)

    @pl.loop(0, _GROWS_W)
    def _(j):
        pltpu.sync_copy(h2_hbm.at[ibuf.at[j]], rows)
        pltpu.sync_copy(rows, out_hbm.at[pl.ds(wid * _GROWS_W * K + j * K, K)])


def _sc_final_gather(h2, sd_idx):
    f = pl.kernel(
        _final_gather_body,
        out_type=jax.ShapeDtypeStruct((2 * B, D), jnp.float32),
        mesh=_mesh,
        scratch_types=[
            pltpu.VMEM((_GROWS_W, K), jnp.int32),
            pltpu.VMEM((K, D), jnp.float32),
        ],
    )
    return f(h2, sd_idx)


BLK = 1024


def _tca_body(degp_ref, emb_ref, tab_ref, norms_ref):
    deg = degp_ref[0] + degp_ref[1]
    norms = lax.rsqrt(jnp.maximum(deg, 1.0))
    norms_ref[...] = norms
    emb = emb_ref[...]
    for r in range(3):
        tab_ref[r] = emb * norms[2 * r][:, None]


def _tc_scale_emb(degp, emb_pad):
    grid = NP // BLK
    return pl.pallas_call(
        _tca_body,
        grid=(grid,),
        in_specs=[
            pl.BlockSpec((2, 6, BLK), lambda i: (0, 0, i)),
            pl.BlockSpec((BLK, D), lambda i: (i, 0)),
        ],
        out_specs=[
            pl.BlockSpec((3, BLK, D), lambda i: (0, i, 0)),
            pl.BlockSpec((6, BLK), lambda i: (0, i)),
        ],
        out_shape=[
            jax.ShapeDtypeStruct((3, NP, D), jnp.float32),
            jax.ShapeDtypeStruct((6, NP), jnp.float32),
        ],
    )(degp, emb_pad)


def _tcb_body(part_ref, norms_ref, w_ref, b_ref, tab2_ref, make_tables):
    bsum = jnp.sum(b_ref[...], axis=0)
    acc = jnp.broadcast_to(bsum[None, :], (BLK, D))
    for r in range(3):
        x = (part_ref[0, r] + part_ref[1, r]) * norms_ref[2 * r + 1][:, None]
        acc = acc + jnp.dot(x, w_ref[r], preferred_element_type=jnp.float32)
    h = jnp.tanh(acc)
    if make_tables:
        for r in range(3):
            tab2_ref[r] = h * norms_ref[2 * r][:, None]
    else:
        tab2_ref[...] = h


def _tc_combine(parts, norms, Ws, bs, make_tables):
    grid = NP // BLK
    if make_tables:
        out_spec = pl.BlockSpec((3, BLK, D), lambda i: (0, i, 0))
        out_shape = jax.ShapeDtypeStruct((3, NP, D), jnp.float32)
    else:
        out_spec = pl.BlockSpec((BLK, D), lambda i: (i, 0))
        out_shape = jax.ShapeDtypeStruct((NP, D), jnp.float32)
    return pl.pallas_call(
        functools.partial(_tcb_body, make_tables=make_tables),
        grid=(grid,),
        in_specs=[
            pl.BlockSpec((2, 3, BLK, D), lambda i: (0, 0, i, 0)),
            pl.BlockSpec((6, BLK), lambda i: (0, i)),
            pl.BlockSpec((3, D, D), lambda i: (0, 0, 0)),
            pl.BlockSpec((3, D), lambda i: (0, 0)),
        ],
        out_specs=out_spec,
        out_shape=out_shape,
    )(parts, norms, Ws, bs)


def _pad_idx(a, off):
    a = a.astype(jnp.int32)
    pad = jnp.full((EPAD - E,), DUMP, jnp.int32)
    return jnp.concatenate([a, pad]) + off


def kernel(edge_index_r0, edge_index_r1, edge_index_r2, src, dst, emb,
           W1_r0, b1_r0, W1_r1, b1_r1, W1_r2, b1_r2,
           W2_r0, b2_r0, W2_r1, b2_r1, W2_r2, b2_r2):
    rels = [edge_index_r0, edge_index_r1, edge_index_r2]

    sidx = jnp.concatenate(
        [_pad_idx(e[0], r * NP) for r, e in enumerate(rels)]).reshape(3 * ROWS_REL, K)
    didx = jnp.concatenate(
        [_pad_idx(e[1], 0) for e in rels]).reshape(3 * ROWS_REL, K)
    deg_idx = jnp.concatenate(
        [_pad_idx(e[i], (2 * r + i) * NP) for r, e in enumerate(rels)
         for i in (0, 1)]).reshape(NW * DEG_ROWS_W, K)
    sd_idx = jnp.concatenate(
        [src.astype(jnp.int32), dst.astype(jnp.int32)]).reshape((2 * B) // K, K)
    emb_pad = jnp.pad(emb, ((0, NP - N), (0, 0)))
    W1s = jnp.stack([W1_r0, W1_r1, W1_r2])
    b1s = jnp.stack([b1_r0, b1_r1, b1_r2])
    W2s = jnp.stack([W2_r0, W2_r1, W2_r2])
    b2s = jnp.stack([b2_r0, b2_r1, b2_r2])

    degp = _sc_degrees(deg_idx)[:, 0].reshape(NC, 6, NP)
    tab1, norms = _tc_scale_emb(degp, emb_pad)

    part1 = _sc_edge_pass(tab1.reshape(3 * NP, D), sidx, didx)
    tab2 = _tc_combine(part1.reshape(NC, 3, NP, D), norms, W1s, b1s, True)

    part2 = _sc_edge_pass(tab2.reshape(3 * NP, D), sidx, didx)
    h2 = _tc_combine(part2.reshape(NC, 3, NP, D), norms, W2s, b2s, False)

    out = _sc_final_gather(h2, sd_idx)
    return (out[:B], out[B:])


# speedup vs baseline: 2.3427x; 2.3427x over previous
import dataclasses
import functools

import jax
import jax.numpy as jnp
from jax import lax
from jax.experimental import pallas as pl
from jax.experimental.pallas import tpu as pltpu
from jax.experimental.pallas import tpu_sc as plsc

N = 10000
D = 128
E = 100000
B = 8192

NP = 10240
DUMP = N
NC, NS, NW = 2, 16, 32
K = 128
ROWS_W = 25
EPAD = NW * ROWS_W * K
ROWS_REL = NW * ROWS_W
DEG_ROWS_W = 6 * ROWS_W
RPS = NP // NS
DRPS = 6 * NP // NS

_mesh = plsc.VectorSubcoreMesh(core_axis_name="c", subcore_axis_name="s",
                               num_cores=NC, num_subcores=NS)

_sc_params = pltpu.CompilerParams()
if "needs_layout_passes" in pltpu.CompilerParams.__dataclass_fields__:
    _sc_params = dataclasses.replace(_sc_params, needs_layout_passes=False)


def _zfill_f32(ref, nrows, ncols16):
    @pl.loop(0, nrows)
    def _(i):
        for c in range(ncols16):
            ref[i, pl.ds(c * 16, 16)] = jnp.zeros((16,), jnp.float32)


def _deg_kernel_body(didx_hbm, out_hbm, ibuf, deg_v):
    cid = lax.axis_index("c")
    sid = lax.axis_index("s")
    wid = cid * NS + sid

    pltpu.sync_copy(didx_hbm.at[wid], ibuf)
    ones = jnp.ones((16,), jnp.float32)

    for a in range(6):
        @pl.loop(0, NP // 16)
        def _(i):
            deg_v[pl.ds(i * 16, 16)] = jnp.zeros((16,), jnp.float32)

        @pl.loop(0, ROWS_W)
        def _(j):
            for c in range(K // 16):
                idx = ibuf[a * ROWS_W + j, pl.ds(c * 16, 16)]
                plsc.addupdate_scatter(deg_v, [idx], ones)

        pltpu.sync_copy(deg_v, out_hbm.at[pl.ds((a * NW + wid) * NP, NP)])


def _sc_degrees(deg_idx):
    f = pl.kernel(
        _deg_kernel_body,
        out_type=jax.ShapeDtypeStruct((6 * NW * NP,), jnp.float32),
        mesh=_mesh,
        scratch_types=[
            pltpu.VMEM((DEG_ROWS_W, K), jnp.int32),
            pltpu.VMEM((NP,), jnp.float32),
        ],
        compiler_params=_sc_params,
    )
    return f(deg_idx)


def _edge_kernel_body(tab_hbm, sidx_hbm, didx_hbm, out_hbm,
                      acc, sbuf, dbuf, rows, zv):
    cid = lax.axis_index("c")
    sid = lax.axis_index("s")
    wid = cid * NS + sid

    _zfill_f32(zv, K, D // 16)

    for r in range(3):
        @pl.loop(0, RPS // K)
        def _(t):
            pltpu.sync_copy(zv, acc.at[pl.ds(sid * RPS + t * K, K)])
        plsc.subcore_barrier()

        base = r * NW + wid
        pltpu.sync_copy(sidx_hbm.at[base], sbuf)
        pltpu.sync_copy(didx_hbm.at[base], dbuf)

        @pl.loop(0, ROWS_W)
        def _(j):
            pltpu.sync_copy(tab_hbm.at[sbuf.at[j]], rows)
            pltpu.sync_copy(rows, acc.at[dbuf.at[j]], add=True)
        plsc.subcore_barrier()

        pltpu.sync_copy(acc.at[pl.ds(sid * RPS, RPS)],
                        out_hbm.at[pl.ds((cid * 3 + r) * NP + sid * RPS, RPS)])
        plsc.subcore_barrier()


def _sc_edge_pass(tab3, sidx, didx):
    f = pl.kernel(
        _edge_kernel_body,
        out_type=jax.ShapeDtypeStruct((NC * 3 * NP, D), jnp.float32),
        mesh=_mesh,
        scratch_types=[
            pltpu.VMEM_SHARED((NP, D), jnp.float32),
            pltpu.VMEM((ROWS_W, K), jnp.int32),
            pltpu.VMEM((ROWS_W, K), jnp.int32),
            pltpu.VMEM((K, D), jnp.float32),
            pltpu.VMEM((K, D), jnp.float32),
        ],
    )
    return f(tab3, sidx, didx)


_GROWS_W = (2 * B) // K // NW


def _final_gather_body(h2_hbm, idx_hbm, out_hbm, ibuf, rows):
    wid = lax.axis_index("c") * NS + lax.axis_index("s")
    pltpu.sync_copy(idx_hbm.at[wid], ibuf)

    @pl.loop(0, _GROWS_W)
    def _(j):
        pltpu.sync_copy(h2_hbm.at[ibuf.at[j]], rows)
        pltpu.sync_copy(rows, out_hbm.at[pl.ds(wid * _GROWS_W * K + j * K, K)])


def _sc_final_gather(h2, sd_idx):
    f = pl.kernel(
        _final_gather_body,
        out_type=jax.ShapeDtypeStruct((2 * B, D), jnp.float32),
        mesh=_mesh,
        scratch_types=[
            pltpu.VMEM((_GROWS_W, K), jnp.int32),
            pltpu.VMEM((K, D), jnp.float32),
        ],
    )
    return f(h2, sd_idx)


BLK = 1024


def _tca_body(degp_ref, emb_ref, tab_ref, norms_ref):
    deg = jnp.sum(degp_ref[...], axis=1)
    norms = lax.rsqrt(jnp.maximum(deg, 1.0))
    norms_ref[...] = norms
    emb = emb_ref[...]
    for r in range(3):
        tab_ref[r] = emb * norms[2 * r][:, None]


def _tc_scale_emb(degp, emb_pad):
    grid = NP // BLK
    return pl.pallas_call(
        _tca_body,
        grid=(grid,),
        in_specs=[
            pl.BlockSpec((6, NW, BLK), lambda i: (0, 0, i)),
            pl.BlockSpec((BLK, D), lambda i: (i, 0)),
        ],
        out_specs=[
            pl.BlockSpec((3, BLK, D), lambda i: (0, i, 0)),
            pl.BlockSpec((6, BLK), lambda i: (0, i)),
        ],
        out_shape=[
            jax.ShapeDtypeStruct((3, NP, D), jnp.float32),
            jax.ShapeDtypeStruct((6, NP), jnp.float32),
        ],
    )(degp, emb_pad)


def _tcb_body(part_ref, norms_ref, w_ref, b_ref, tab2_ref, make_tables):
    bsum = jnp.sum(b_ref[...], axis=0)
    acc = jnp.broadcast_to(bsum[None, :], (BLK, D))
    for r in range(3):
        x = (part_ref[0, r] + part_ref[1, r]) * norms_ref[2 * r + 1][:, None]
        acc = acc + jnp.dot(x, w_ref[r], preferred_element_type=jnp.float32)
    h = jnp.tanh(acc)
    if make_tables:
        for r in range(3):
            tab2_ref[r] = h * norms_ref[2 * r][:, None]
    else:
        tab2_ref[...] = h


def _tc_combine(parts, norms, Ws, bs, make_tables):
    grid = NP // BLK
    if make_tables:
        out_spec = pl.BlockSpec((3, BLK, D), lambda i: (0, i, 0))
        out_shape = jax.ShapeDtypeStruct((3, NP, D), jnp.float32)
    else:
        out_spec = pl.BlockSpec((BLK, D), lambda i: (i, 0))
        out_shape = jax.ShapeDtypeStruct((NP, D), jnp.float32)
    return pl.pallas_call(
        functools.partial(_tcb_body, make_tables=make_tables),
        grid=(grid,),
        in_specs=[
            pl.BlockSpec((2, 3, BLK, D), lambda i: (0, 0, i, 0)),
            pl.BlockSpec((6, BLK), lambda i: (0, i)),
            pl.BlockSpec((3, D, D), lambda i: (0, 0, 0)),
            pl.BlockSpec((3, D), lambda i: (0, 0)),
        ],
        out_specs=out_spec,
        out_shape=out_shape,
    )(parts, norms, Ws, bs)


def _pad_idx(a, off):
    a = a.astype(jnp.int32)
    pad = jnp.full((EPAD - E,), DUMP, jnp.int32)
    return jnp.concatenate([a, pad]) + off


def kernel(edge_index_r0, edge_index_r1, edge_index_r2, src, dst, emb,
           W1_r0, b1_r0, W1_r1, b1_r1, W1_r2, b1_r2,
           W2_r0, b2_r0, W2_r1, b2_r1, W2_r2, b2_r2):
    rels = [edge_index_r0, edge_index_r1, edge_index_r2]

    sidx = jnp.concatenate(
        [_pad_idx(e[0], r * NP) for r, e in enumerate(rels)]).reshape(
            3 * NW, ROWS_W, K)
    didx = jnp.concatenate(
        [_pad_idx(e[1], 0) for e in rels]).reshape(3 * NW, ROWS_W, K)
    deg_idx = jnp.concatenate(
        [_pad_idx(e[i], 0) for e in rels for i in (0, 1)])
    deg_idx = deg_idx.reshape(6, NW, ROWS_W, K).transpose(1, 0, 2, 3).reshape(
        NW, DEG_ROWS_W, K)
    sd_idx = jnp.concatenate(
        [src.astype(jnp.int32), dst.astype(jnp.int32)]).reshape(
            NW, _GROWS_W, K)
    emb_pad = jnp.pad(emb, ((0, NP - N), (0, 0)))
    W1s = jnp.stack([W1_r0, W1_r1, W1_r2])
    b1s = jnp.stack([b1_r0, b1_r1, b1_r2])
    W2s = jnp.stack([W2_r0, W2_r1, W2_r2])
    b2s = jnp.stack([b2_r0, b2_r1, b2_r2])

    degp = _sc_degrees(deg_idx).reshape(6, NW, NP)
    tab1, norms = _tc_scale_emb(degp, emb_pad)

    part1 = _sc_edge_pass(tab1.reshape(3 * NP, D), sidx, didx)
    tab2 = _tc_combine(part1.reshape(NC, 3, NP, D), norms, W1s, b1s, True)

    part2 = _sc_edge_pass(tab2.reshape(3 * NP, D), sidx, didx)
    h2 = _tc_combine(part2.reshape(NC, 3, NP, D), norms, W2s, b2s, False)

    out = _sc_final_gather(h2, sd_idx)
    return (out[:B], out[B:])


# speedup vs baseline: 2.6010x; 1.1102x over previous
import dataclasses
import functools

import jax
import jax.numpy as jnp
from jax import lax
from jax.experimental import pallas as pl
from jax.experimental.pallas import tpu as pltpu
from jax.experimental.pallas import tpu_sc as plsc

N = 10000
D = 128
E = 100000
B = 8192

NP = 10240
DUMP = N
NC, NS, NW = 2, 16, 32
K = 128
ROWS_W = 25
EPAD = NW * ROWS_W * K
ROWS_REL = NW * ROWS_W
DEG_ROWS_W = 6 * ROWS_W
RPS = NP // NS
DRPS = 6 * NP // NS

_mesh = plsc.VectorSubcoreMesh(core_axis_name="c", subcore_axis_name="s",
                               num_cores=NC, num_subcores=NS)

_sc_params = pltpu.CompilerParams()
if "needs_layout_passes" in pltpu.CompilerParams.__dataclass_fields__:
    _sc_params = dataclasses.replace(_sc_params, needs_layout_passes=False)


def _zfill_f32(ref, nrows, ncols16):
    @pl.loop(0, nrows)
    def _(i):
        for c in range(ncols16):
            ref[i, pl.ds(c * 16, 16)] = jnp.zeros((16,), jnp.float32)


def _deg_kernel_body(didx_hbm, out_hbm, ibuf, deg_v):
    cid = lax.axis_index("c")
    sid = lax.axis_index("s")
    wid = cid * NS + sid

    pltpu.sync_copy(didx_hbm.at[wid], ibuf)
    ones = jnp.ones((16,), jnp.float32)

    for a in range(6):
        @pl.loop(0, NP // 16)
        def _(i):
            deg_v[pl.ds(i * 16, 16)] = jnp.zeros((16,), jnp.float32)

        @pl.loop(0, ROWS_W)
        def _(j):
            for c in range(K // 16):
                idx = ibuf[a * ROWS_W + j, pl.ds(c * 16, 16)]
                plsc.addupdate_scatter(deg_v, [idx], ones)

        pltpu.sync_copy(deg_v, out_hbm.at[pl.ds((a * NW + wid) * NP, NP)])


def _sc_degrees(deg_idx):
    f = pl.kernel(
        _deg_kernel_body,
        out_type=jax.ShapeDtypeStruct((6 * NW * NP,), jnp.float32),
        mesh=_mesh,
        scratch_types=[
            pltpu.VMEM((DEG_ROWS_W, K), jnp.int32),
            pltpu.VMEM((NP,), jnp.float32),
        ],
        compiler_params=_sc_params,
    )
    return f(deg_idx)


NBUF = 2
ZROWS = 32


def _edge_kernel_body(tab_hbm, sidx_hbm, didx_hbm, out_hbm,
                      acc, sbuf, dbuf, rows, zv, gsem, ssem):
    cid = lax.axis_index("c")
    sid = lax.axis_index("s")
    wid = cid * NS + sid

    _zfill_f32(zv, ZROWS, D // 16)

    for r in range(3):
        @pl.loop(0, RPS // ZROWS)
        def _(t):
            pltpu.sync_copy(zv, acc.at[pl.ds(sid * RPS + t * ZROWS, ZROWS)])
        plsc.subcore_barrier()

        base = r * NW + wid
        pltpu.sync_copy(sidx_hbm.at[base], sbuf)
        pltpu.sync_copy(didx_hbm.at[base], dbuf)

        gathers = {}
        scatters = {}
        for j in range(min(NBUF, ROWS_W)):
            b = j % NBUF
            gathers[j] = pltpu.async_copy(
                tab_hbm.at[sbuf.at[j]], rows.at[pl.ds(b * K, K)], gsem.at[b])
        for j in range(ROWS_W):
            b = j % NBUF
            gathers[j].wait()
            scatters[j] = pltpu.async_copy(
                rows.at[pl.ds(b * K, K)], acc.at[dbuf.at[j]], ssem.at[b],
                add=True)
            nj = j + NBUF
            if nj < ROWS_W:
                scatters[j].wait()
                gathers[nj] = pltpu.async_copy(
                    tab_hbm.at[sbuf.at[nj]], rows.at[pl.ds(b * K, K)],
                    gsem.at[b])
        for j in range(max(0, ROWS_W - NBUF), ROWS_W):
            scatters[j].wait()
        plsc.subcore_barrier()

        pltpu.sync_copy(acc.at[pl.ds(sid * RPS, RPS)],
                        out_hbm.at[pl.ds((cid * 3 + r) * NP + sid * RPS, RPS)])
        plsc.subcore_barrier()


def _sc_edge_pass(tab3, sidx, didx):
    f = pl.kernel(
        _edge_kernel_body,
        out_type=jax.ShapeDtypeStruct((NC * 3 * NP, D), jnp.float32),
        mesh=_mesh,
        scratch_types=[
            pltpu.VMEM_SHARED((NP, D), jnp.float32),
            pltpu.VMEM((ROWS_W, K), jnp.int32),
            pltpu.VMEM((ROWS_W, K), jnp.int32),
            pltpu.VMEM((NBUF * K, D), jnp.float32),
            pltpu.VMEM((ZROWS, D), jnp.float32),
            pltpu.SemaphoreType.DMA((NBUF,)),
            pltpu.SemaphoreType.DMA((NBUF,)),
        ],
    )
    return f(tab3, sidx, didx)


_GROWS_W = (2 * B) // K // NW


def _final_gather_body(h2_hbm, idx_hbm, out_hbm, ibuf, rows):
    wid = lax.axis_index("c") * NS + lax.axis_index("s")
    pltpu.sync_copy(idx_hbm.at[wid], ibuf)

    @pl.loop(0, _GROWS_W)
    def _(j):
        pltpu.sync_copy(h2_hbm.at[ibuf.at[j]], rows)
        pltpu.sync_copy(rows, out_hbm.at[pl.ds(wid * _GROWS_W * K + j * K, K)])


def _sc_final_gather(h2, sd_idx):
    f = pl.kernel(
        _final_gather_body,
        out_type=jax.ShapeDtypeStruct((2 * B, D), jnp.float32),
        mesh=_mesh,
        scratch_types=[
            pltpu.VMEM((_GROWS_W, K), jnp.int32),
            pltpu.VMEM((K, D), jnp.float32),
        ],
    )
    return f(h2, sd_idx)


BLK = 1024


def _tca_body(degp_ref, emb_ref, tab_ref, norms_ref):
    deg = jnp.sum(degp_ref[...], axis=1)
    norms = lax.rsqrt(jnp.maximum(deg, 1.0))
    norms_ref[...] = norms
    emb = emb_ref[...]
    for r in range(3):
        tab_ref[r] = emb * norms[2 * r][:, None]


def _tc_scale_emb(degp, emb_pad):
    grid = NP // BLK
    return pl.pallas_call(
        _tca_body,
        grid=(grid,),
        in_specs=[
            pl.BlockSpec((6, NW, BLK), lambda i: (0, 0, i)),
            pl.BlockSpec((BLK, D), lambda i: (i, 0)),
        ],
        out_specs=[
            pl.BlockSpec((3, BLK, D), lambda i: (0, i, 0)),
            pl.BlockSpec((6, BLK), lambda i: (0, i)),
        ],
        out_shape=[
            jax.ShapeDtypeStruct((3, NP, D), jnp.float32),
            jax.ShapeDtypeStruct((6, NP), jnp.float32),
        ],
    )(degp, emb_pad)


def _tcb_body(part_ref, norms_ref, w_ref, b_ref, tab2_ref, make_tables):
    bsum = jnp.sum(b_ref[...], axis=0)
    acc = jnp.broadcast_to(bsum[None, :], (BLK, D))
    for r in range(3):
        x = (part_ref[0, r] + part_ref[1, r]) * norms_ref[2 * r + 1][:, None]
        acc = acc + jnp.dot(x, w_ref[r], preferred_element_type=jnp.float32)
    h = jnp.tanh(acc)
    if make_tables:
        for r in range(3):
            tab2_ref[r] = h * norms_ref[2 * r][:, None]
    else:
        tab2_ref[...] = h


def _tc_combine(parts, norms, Ws, bs, make_tables):
    grid = NP // BLK
    if make_tables:
        out_spec = pl.BlockSpec((3, BLK, D), lambda i: (0, i, 0))
        out_shape = jax.ShapeDtypeStruct((3, NP, D), jnp.float32)
    else:
        out_spec = pl.BlockSpec((BLK, D), lambda i: (i, 0))
        out_shape = jax.ShapeDtypeStruct((NP, D), jnp.float32)
    return pl.pallas_call(
        functools.partial(_tcb_body, make_tables=make_tables),
        grid=(grid,),
        in_specs=[
            pl.BlockSpec((2, 3, BLK, D), lambda i: (0, 0, i, 0)),
            pl.BlockSpec((6, BLK), lambda i: (0, i)),
            pl.BlockSpec((3, D, D), lambda i: (0, 0, 0)),
            pl.BlockSpec((3, D), lambda i: (0, 0)),
        ],
        out_specs=out_spec,
        out_shape=out_shape,
    )(parts, norms, Ws, bs)


def _pad_idx(a, off):
    a = a.astype(jnp.int32)
    pad = jnp.full((EPAD - E,), DUMP, jnp.int32)
    return jnp.concatenate([a, pad]) + off


def kernel(edge_index_r0, edge_index_r1, edge_index_r2, src, dst, emb,
           W1_r0, b1_r0, W1_r1, b1_r1, W1_r2, b1_r2,
           W2_r0, b2_r0, W2_r1, b2_r1, W2_r2, b2_r2):
    rels = [edge_index_r0, edge_index_r1, edge_index_r2]

    sidx = jnp.concatenate(
        [_pad_idx(e[0], r * NP) for r, e in enumerate(rels)]).reshape(
            3 * NW, ROWS_W, K)
    didx = jnp.concatenate(
        [_pad_idx(e[1], 0) for e in rels]).reshape(3 * NW, ROWS_W, K)
    deg_idx = jnp.concatenate(
        [_pad_idx(e[i], 0) for e in rels for i in (0, 1)])
    deg_idx = deg_idx.reshape(6, NW, ROWS_W, K).transpose(1, 0, 2, 3).reshape(
        NW, DEG_ROWS_W, K)
    sd_idx = jnp.concatenate(
        [src.astype(jnp.int32), dst.astype(jnp.int32)]).reshape(
            NW, _GROWS_W, K)
    emb_pad = jnp.pad(emb, ((0, NP - N), (0, 0)))
    W1s = jnp.stack([W1_r0, W1_r1, W1_r2])
    b1s = jnp.stack([b1_r0, b1_r1, b1_r2])
    W2s = jnp.stack([W2_r0, W2_r1, W2_r2])
    b2s = jnp.stack([b2_r0, b2_r1, b2_r2])

    degp = _sc_degrees(deg_idx).reshape(6, NW, NP)
    tab1, norms = _tc_scale_emb(degp, emb_pad)

    part1 = _sc_edge_pass(tab1.reshape(3 * NP, D), sidx, didx)
    tab2 = _tc_combine(part1.reshape(NC, 3, NP, D), norms, W1s, b1s, True)

    part2 = _sc_edge_pass(tab2.reshape(3 * NP, D), sidx, didx)
    h2 = _tc_combine(part2.reshape(NC, 3, NP, D), norms, W2s, b2s, False)

    out = _sc_final_gather(h2, sd_idx)
    return (out[:B], out[B:])


# speedup vs baseline: 6.6806x; 2.5685x over previous
import dataclasses
import functools

import jax
import jax.numpy as jnp
from jax import lax
from jax.experimental import pallas as pl
from jax.experimental.pallas import tpu as pltpu
from jax.experimental.pallas import tpu_sc as plsc

N = 10000
D = 128
E = 100000
B = 8192

NP = 10240
DUMP = N
NC, NS, NW = 2, 16, 32
K = 128
ROWS_W = 25
EPAD = NW * ROWS_W * K
ROWS_REL = NW * ROWS_W
DEG_ROWS_W = 6 * ROWS_W
RPS = NP // NS
DRPS = 6 * NP // NS

_mesh = plsc.VectorSubcoreMesh(core_axis_name="c", subcore_axis_name="s",
                               num_cores=NC, num_subcores=NS)

_sc_params = pltpu.CompilerParams()
if "needs_layout_passes" in pltpu.CompilerParams.__dataclass_fields__:
    _sc_params = dataclasses.replace(_sc_params, needs_layout_passes=False)


def _zfill_f32(ref, nrows, ncols16):
    @pl.loop(0, nrows)
    def _(i):
        for c in range(ncols16):
            ref[i, pl.ds(c * 16, 16)] = jnp.zeros((16,), jnp.float32)


def _deg_kernel_body(didx_hbm, out_hbm, ibuf, deg_v):
    cid = lax.axis_index("c")
    sid = lax.axis_index("s")
    wid = cid * NS + sid

    pltpu.sync_copy(didx_hbm.at[wid], ibuf)
    ones = jnp.ones((16,), jnp.float32)

    for a in range(6):
        @pl.loop(0, NP // 16)
        def _(i):
            deg_v[pl.ds(i * 16, 16)] = jnp.zeros((16,), jnp.float32)

        @pl.loop(0, ROWS_W)
        def _(j):
            for c in range(K // 16):
                idx = ibuf[a * ROWS_W + j, pl.ds(c * 16, 16)]
                plsc.addupdate_scatter(deg_v, [idx], ones)

        pltpu.sync_copy(deg_v, out_hbm.at[pl.ds((a * NW + wid) * NP, NP)])


def _sc_degrees(deg_idx):
    f = pl.kernel(
        _deg_kernel_body,
        out_type=jax.ShapeDtypeStruct((6 * NW * NP,), jnp.float32),
        mesh=_mesh,
        scratch_types=[
            pltpu.VMEM((DEG_ROWS_W, K), jnp.int32),
            pltpu.VMEM((NP,), jnp.float32),
        ],
        compiler_params=_sc_params,
    )
    return f(deg_idx)


NBUF = 2
ZROWS = 32


def _edge_kernel_body(tab_hbm, sidx_hbm, didx_hbm, out_hbm,
                      acc, sbuf, dbuf, rows, zv, gsem, ssem):
    cid = lax.axis_index("c")
    sid = lax.axis_index("s")
    wid = cid * NS + sid

    _zfill_f32(zv, ZROWS, D // 16)

    for r in range(3):
        @pl.loop(0, RPS // ZROWS)
        def _(t):
            pltpu.sync_copy(zv, acc.at[pl.ds(sid * RPS + t * ZROWS, ZROWS)])
        plsc.subcore_barrier()

        base = r * NW + wid
        pltpu.sync_copy(sidx_hbm.at[base], sbuf)
        pltpu.sync_copy(didx_hbm.at[base], dbuf)

        gathers = {}
        scatters = {}
        for j in range(min(NBUF, ROWS_W)):
            b = j % NBUF
            gathers[j] = pltpu.async_copy(
                tab_hbm.at[sbuf.at[j]], rows.at[pl.ds(b * K, K)], gsem.at[b])
        for j in range(ROWS_W):
            b = j % NBUF
            gathers[j].wait()
            scatters[j] = pltpu.async_copy(
                rows.at[pl.ds(b * K, K)], acc.at[dbuf.at[j]], ssem.at[b],
                add=True)
            nj = j + NBUF
            if nj < ROWS_W:
                scatters[j].wait()
                gathers[nj] = pltpu.async_copy(
                    tab_hbm.at[sbuf.at[nj]], rows.at[pl.ds(b * K, K)],
                    gsem.at[b])
        for j in range(max(0, ROWS_W - NBUF), ROWS_W):
            scatters[j].wait()
        plsc.subcore_barrier()

        pltpu.sync_copy(acc.at[pl.ds(sid * RPS, RPS)],
                        out_hbm.at[pl.ds((cid * 3 + r) * NP + sid * RPS, RPS)])
        plsc.subcore_barrier()


def _sc_edge_pass(tab3, sidx, didx):
    f = pl.kernel(
        _edge_kernel_body,
        out_type=jax.ShapeDtypeStruct((NC * 3 * NP, D), jnp.float32),
        mesh=_mesh,
        scratch_types=[
            pltpu.VMEM_SHARED((NP, D), jnp.float32),
            pltpu.VMEM((ROWS_W, K), jnp.int32),
            pltpu.VMEM((ROWS_W, K), jnp.int32),
            pltpu.VMEM((NBUF * K, D), jnp.float32),
            pltpu.VMEM((ZROWS, D), jnp.float32),
            pltpu.SemaphoreType.DMA((NBUF,)),
            pltpu.SemaphoreType.DMA((NBUF,)),
        ],
    )
    return f(tab3, sidx, didx)


_GROWS_W = (2 * B) // K // NW


def _final_gather_body(h2_hbm, idx_hbm, out_hbm, ibuf, rows):
    wid = lax.axis_index("c") * NS + lax.axis_index("s")
    pltpu.sync_copy(idx_hbm.at[wid], ibuf)

    @pl.loop(0, _GROWS_W)
    def _(j):
        pltpu.sync_copy(h2_hbm.at[ibuf.at[j]], rows)
        pltpu.sync_copy(rows, out_hbm.at[pl.ds(wid * _GROWS_W * K + j * K, K)])


def _sc_final_gather(h2, sd_idx):
    f = pl.kernel(
        _final_gather_body,
        out_type=jax.ShapeDtypeStruct((2 * B, D), jnp.float32),
        mesh=_mesh,
        scratch_types=[
            pltpu.VMEM((_GROWS_W, K), jnp.int32),
            pltpu.VMEM((K, D), jnp.float32),
        ],
    )
    return f(h2, sd_idx)


BLK = 1024


def _tca_body(degp_ref, emb_ref, tab_ref, norms_ref):
    deg = jnp.sum(degp_ref[...], axis=1)
    norms = lax.rsqrt(jnp.maximum(deg, 1.0))
    norms_ref[...] = norms
    emb = emb_ref[...]
    for r in range(3):
        tab_ref[r] = emb * norms[2 * r][:, None]


def _tc_scale_emb(degp, emb_pad):
    grid = NP // BLK
    return pl.pallas_call(
        _tca_body,
        grid=(grid,),
        in_specs=[
            pl.BlockSpec((6, NW, BLK), lambda i: (0, 0, i)),
            pl.BlockSpec((BLK, D), lambda i: (i, 0)),
        ],
        out_specs=[
            pl.BlockSpec((3, BLK, D), lambda i: (0, i, 0)),
            pl.BlockSpec((6, BLK), lambda i: (0, i)),
        ],
        out_shape=[
            jax.ShapeDtypeStruct((3, NP, D), jnp.float32),
            jax.ShapeDtypeStruct((6, NP), jnp.float32),
        ],
    )(degp, emb_pad)


def _tcb_body(part_ref, norms_ref, w_ref, b_ref, tab2_ref, make_tables):
    bsum = jnp.sum(b_ref[...], axis=0)
    acc = jnp.broadcast_to(bsum[None, :], (BLK, D))
    for r in range(3):
        x = (part_ref[0, r] + part_ref[1, r]) * norms_ref[2 * r + 1][:, None]
        acc = acc + jnp.dot(x, w_ref[r], preferred_element_type=jnp.float32)
    h = jnp.tanh(acc)
    if make_tables:
        for r in range(3):
            tab2_ref[r] = h * norms_ref[2 * r][:, None]
    else:
        tab2_ref[...] = h


def _tc_combine(parts, norms, Ws, bs, make_tables):
    grid = NP // BLK
    if make_tables:
        out_spec = pl.BlockSpec((3, BLK, D), lambda i: (0, i, 0))
        out_shape = jax.ShapeDtypeStruct((3, NP, D), jnp.float32)
    else:
        out_spec = pl.BlockSpec((BLK, D), lambda i: (i, 0))
        out_shape = jax.ShapeDtypeStruct((NP, D), jnp.float32)
    return pl.pallas_call(
        functools.partial(_tcb_body, make_tables=make_tables),
        grid=(grid,),
        in_specs=[
            pl.BlockSpec((2, 3, BLK, D), lambda i: (0, 0, i, 0)),
            pl.BlockSpec((6, BLK), lambda i: (0, i)),
            pl.BlockSpec((3, D, D), lambda i: (0, 0, 0)),
            pl.BlockSpec((3, D), lambda i: (0, 0)),
        ],
        out_specs=out_spec,
        out_shape=out_shape,
    )(parts, norms, Ws, bs)


def _pad_idx(a, off):
    a = a.astype(jnp.int32)
    pad = N + (jnp.arange(EPAD - E, dtype=jnp.int32) % (NP - N))
    return jnp.concatenate([a, pad]) + off


def kernel(edge_index_r0, edge_index_r1, edge_index_r2, src, dst, emb,
           W1_r0, b1_r0, W1_r1, b1_r1, W1_r2, b1_r2,
           W2_r0, b2_r0, W2_r1, b2_r1, W2_r2, b2_r2):
    rels = [edge_index_r0, edge_index_r1, edge_index_r2]

    sidx = jnp.concatenate(
        [_pad_idx(e[0], r * NP) for r, e in enumerate(rels)]).reshape(
            3 * NW, ROWS_W, K)
    didx = jnp.concatenate(
        [_pad_idx(e[1], 0) for e in rels]).reshape(3 * NW, ROWS_W, K)
    deg_idx = jnp.concatenate(
        [_pad_idx(e[i], 0) for e in rels for i in (0, 1)])
    deg_idx = deg_idx.reshape(6, NW, ROWS_W, K).transpose(1, 0, 2, 3).reshape(
        NW, DEG_ROWS_W, K)
    sd_idx = jnp.concatenate(
        [src.astype(jnp.int32), dst.astype(jnp.int32)]).reshape(
            NW, _GROWS_W, K)
    emb_pad = jnp.pad(emb, ((0, NP - N), (0, 0)))
    W1s = jnp.stack([W1_r0, W1_r1, W1_r2])
    b1s = jnp.stack([b1_r0, b1_r1, b1_r2])
    W2s = jnp.stack([W2_r0, W2_r1, W2_r2])
    b2s = jnp.stack([b2_r0, b2_r1, b2_r2])

    degp = _sc_degrees(deg_idx).reshape(6, NW, NP)
    tab1, norms = _tc_scale_emb(degp, emb_pad)

    part1 = _sc_edge_pass(tab1.reshape(3 * NP, D), sidx, didx)
    tab2 = _tc_combine(part1.reshape(NC, 3, NP, D), norms, W1s, b1s, True)

    part2 = _sc_edge_pass(tab2.reshape(3 * NP, D), sidx, didx)
    h2 = _tc_combine(part2.reshape(NC, 3, NP, D), norms, W2s, b2s, False)

    out = _sc_final_gather(h2, sd_idx)
    return (out[:B], out[B:])


# speedup vs baseline: 7.0601x; 1.0568x over previous
import dataclasses
import functools

import jax
import jax.numpy as jnp
from jax import lax
from jax.experimental import pallas as pl
from jax.experimental.pallas import tpu as pltpu
from jax.experimental.pallas import tpu_sc as plsc

N = 10000
D = 128
E = 100000
B = 8192

NP = 10240
NC, NS, NW = 2, 16, 32
K = 128
ROWS_W = 25
EPAD = NW * ROWS_W * K
RPS = NP // NS
NBUF = 2
ZROWS = 32

_mesh = plsc.VectorSubcoreMesh(core_axis_name="c", subcore_axis_name="s",
                               num_cores=NC, num_subcores=NS)

_sc_params = pltpu.CompilerParams()
if "needs_layout_passes" in pltpu.CompilerParams.__dataclass_fields__:
    _sc_params = dataclasses.replace(_sc_params, needs_layout_passes=False)


def _zfill_f32(ref, nrows, ncols16):
    @pl.loop(0, nrows)
    def _(i):
        for c in range(ncols16):
            ref[i, pl.ds(c * 16, 16)] = jnp.zeros((16,), jnp.float32)


def _deg_kernel_body(sidx_hbm, didx_hbm, out_hbm, ibuf, deg_v):
    cid = lax.axis_index("c")
    sid = lax.axis_index("s")
    wid = cid * NS + sid

    pltpu.sync_copy(sidx_hbm.at[wid], ibuf.at[pl.ds(0, ROWS_W)])
    pltpu.sync_copy(didx_hbm.at[wid], ibuf.at[pl.ds(ROWS_W, ROWS_W)])
    ones = jnp.ones((16,), jnp.float32)

    for a in range(2):
        @pl.loop(0, NP // 16)
        def _(i):
            deg_v[pl.ds(i * 16, 16)] = jnp.zeros((16,), jnp.float32)

        @pl.loop(0, ROWS_W)
        def _(j):
            for c in range(K // 16):
                idx = ibuf[a * ROWS_W + j, pl.ds(c * 16, 16)]
                plsc.addupdate_scatter(deg_v, [idx], ones)

        pltpu.sync_copy(deg_v, out_hbm.at[pl.ds((a * NW + wid) * NP, NP)])


def _sc_degrees(sidx_r, didx_r):
    f = pl.kernel(
        _deg_kernel_body,
        out_type=jax.ShapeDtypeStruct((2 * NW * NP,), jnp.float32),
        mesh=_mesh,
        scratch_types=[
            pltpu.VMEM((2 * ROWS_W, K), jnp.int32),
            pltpu.VMEM((NP,), jnp.float32),
        ],
        compiler_params=_sc_params,
    )
    return f(sidx_r, didx_r)


def _edge_kernel_body(tab_hbm, sidx_hbm, didx_hbm, out_hbm,
                      acc, sbuf, dbuf, rows, zv, gsem, ssem):
    cid = lax.axis_index("c")
    sid = lax.axis_index("s")
    wid = cid * NS + sid

    _zfill_f32(zv, ZROWS, D // 16)

    @pl.loop(0, RPS // ZROWS)
    def _(t):
        pltpu.sync_copy(zv, acc.at[pl.ds(sid * RPS + t * ZROWS, ZROWS)])
    plsc.subcore_barrier()

    pltpu.sync_copy(sidx_hbm.at[wid], sbuf)
    pltpu.sync_copy(didx_hbm.at[wid], dbuf)

    gathers = {}
    scatters = {}
    for j in range(min(NBUF, ROWS_W)):
        b = j % NBUF
        gathers[j] = pltpu.async_copy(
            tab_hbm.at[sbuf.at[j]], rows.at[pl.ds(b * K, K)], gsem.at[b])
    for j in range(ROWS_W):
        b = j % NBUF
        gathers[j].wait()
        scatters[j] = pltpu.async_copy(
            rows.at[pl.ds(b * K, K)], acc.at[dbuf.at[j]], ssem.at[b],
            add=True)
        nj = j + NBUF
        if nj < ROWS_W:
            scatters[j].wait()
            gathers[nj] = pltpu.async_copy(
                tab_hbm.at[sbuf.at[nj]], rows.at[pl.ds(b * K, K)],
                gsem.at[b])
    for j in range(max(0, ROWS_W - NBUF), ROWS_W):
        scatters[j].wait()
    plsc.subcore_barrier()

    pltpu.sync_copy(acc.at[pl.ds(sid * RPS, RPS)],
                    out_hbm.at[pl.ds(cid * NP + sid * RPS, RPS)])


def _sc_edge_pass(tab_r, sidx_r, didx_r):
    f = pl.kernel(
        _edge_kernel_body,
        out_type=jax.ShapeDtypeStruct((NC * NP, D), jnp.float32),
        mesh=_mesh,
        scratch_types=[
            pltpu.VMEM_SHARED((NP, D), jnp.float32),
            pltpu.VMEM((ROWS_W, K), jnp.int32),
            pltpu.VMEM((ROWS_W, K), jnp.int32),
            pltpu.VMEM((NBUF * K, D), jnp.float32),
            pltpu.VMEM((ZROWS, D), jnp.float32),
            pltpu.SemaphoreType.DMA((NBUF,)),
            pltpu.SemaphoreType.DMA((NBUF,)),
        ],
    )
    return f(tab_r, sidx_r, didx_r)


_GROWS_W = (2 * B) // K // NW


def _final_gather_body(h2_hbm, idx_hbm, o1_hbm, o2_hbm, ibuf, rows):
    wid = lax.axis_index("c") * NS + lax.axis_index("s")
    pltpu.sync_copy(idx_hbm.at[wid], ibuf)
    half = _GROWS_W // 2

    @pl.loop(0, half)
    def _(j):
        pltpu.sync_copy(h2_hbm.at[ibuf.at[j]], rows)
        pltpu.sync_copy(rows, o1_hbm.at[pl.ds(wid * half * K + j * K, K)])

    @pl.loop(0, half)
    def _(j):
        pltpu.sync_copy(h2_hbm.at[ibuf.at[half + j]], rows)
        pltpu.sync_copy(rows, o2_hbm.at[pl.ds(wid * half * K + j * K, K)])


def _sc_final_gather(h2, sd_idx):
    f = pl.kernel(
        _final_gather_body,
        out_type=[jax.ShapeDtypeStruct((B, D), jnp.float32),
                  jax.ShapeDtypeStruct((B, D), jnp.float32)],
        mesh=_mesh,
        scratch_types=[
            pltpu.VMEM((_GROWS_W, K), jnp.int32),
            pltpu.VMEM((K, D), jnp.float32),
        ],
    )
    return f(h2, sd_idx)


BLK = 1024


def _tca_body(degp_ref, emb_ref, tab_ref, norms_ref):
    deg = jnp.sum(degp_ref[...], axis=1)
    norms = lax.rsqrt(jnp.maximum(deg, 1.0))
    norms_ref[...] = norms
    tab_ref[...] = emb_ref[...] * norms[0][:, None]


def _tc_scale_emb(degp_r, emb_pad):
    return pl.pallas_call(
        _tca_body,
        grid=(NP // BLK,),
        in_specs=[
            pl.BlockSpec((2, NW, BLK), lambda i: (0, 0, i)),
            pl.BlockSpec((BLK, D), lambda i: (i, 0)),
        ],
        out_specs=[
            pl.BlockSpec((BLK, D), lambda i: (i, 0)),
            pl.BlockSpec((2, BLK), lambda i: (0, i)),
        ],
        out_shape=[
            jax.ShapeDtypeStruct((NP, D), jnp.float32),
            jax.ShapeDtypeStruct((2, NP), jnp.float32),
        ],
    )(degp_r, emb_pad)


def _tcy_body(part_ref, norms_ref, w_ref, y_ref):
    x = (part_ref[0] + part_ref[1]) * norms_ref[1][:, None]
    y_ref[...] = jnp.dot(x, w_ref[...], preferred_element_type=jnp.float32)


def _tc_partial_matmul(part_r, norms_r, W_r):
    return pl.pallas_call(
        _tcy_body,
        grid=(NP // BLK,),
        in_specs=[
            pl.BlockSpec((2, BLK, D), lambda i: (0, i, 0)),
            pl.BlockSpec((2, BLK), lambda i: (0, i)),
            pl.BlockSpec((D, D), lambda i: (0, 0)),
        ],
        out_specs=pl.BlockSpec((BLK, D), lambda i: (i, 0)),
        out_shape=jax.ShapeDtypeStruct((NP, D), jnp.float32),
    )(part_r, norms_r, W_r)


def _tcf_tables_body(y0_ref, y1_ref, part_ref, n0_ref, n1_ref, n2_ref,
                     w_ref, b_ref, o0_ref, o1_ref, o2_ref):
    bsum = jnp.sum(b_ref[...], axis=0)
    x = (part_ref[0] + part_ref[1]) * n2_ref[1][:, None]
    acc = (y0_ref[...] + y1_ref[...] + bsum[None, :]
           + jnp.dot(x, w_ref[...], preferred_element_type=jnp.float32))
    h = jnp.tanh(acc)
    o0_ref[...] = h * n0_ref[0][:, None]
    o1_ref[...] = h * n1_ref[0][:, None]
    o2_ref[...] = h * n2_ref[0][:, None]


def _tcf_final_body(y0_ref, y1_ref, part_ref, n0_ref, n1_ref, n2_ref,
                    w_ref, b_ref, o0_ref):
    bsum = jnp.sum(b_ref[...], axis=0)
    x = (part_ref[0] + part_ref[1]) * n2_ref[1][:, None]
    acc = (y0_ref[...] + y1_ref[...] + bsum[None, :]
           + jnp.dot(x, w_ref[...], preferred_element_type=jnp.float32))
    o0_ref[...] = jnp.tanh(acc)


def _tc_combine(y0, y1, part2, n0, n1, n2, W_2, bs, make_tables):
    nblk = pl.BlockSpec((2, BLK), lambda i: (0, i))
    row = pl.BlockSpec((BLK, D), lambda i: (i, 0))
    if make_tables:
        body = _tcf_tables_body
        out_specs = [row, row, row]
        out_shape = [jax.ShapeDtypeStruct((NP, D), jnp.float32)] * 3
    else:
        body = _tcf_final_body
        out_specs = [row]
        out_shape = [jax.ShapeDtypeStruct((NP, D), jnp.float32)]
    return pl.pallas_call(
        body,
        grid=(NP // BLK,),
        in_specs=[
            row, row,
            pl.BlockSpec((2, BLK, D), lambda i: (0, i, 0)),
            nblk, nblk, nblk,
            pl.BlockSpec((D, D), lambda i: (0, 0)),
            pl.BlockSpec((3, D), lambda i: (0, 0)),
        ],
        out_specs=out_specs,
        out_shape=out_shape,
    )(y0, y1, part2, n0, n1, n2, W_2, bs)


def _pad_idx(a):
    a = a.astype(jnp.int32)
    pad = N + (jnp.arange(EPAD - E, dtype=jnp.int32) % (NP - N))
    return jnp.concatenate([a, pad]).reshape(NW, ROWS_W, K)


def kernel(edge_index_r0, edge_index_r1, edge_index_r2, src, dst, emb,
           W1_r0, b1_r0, W1_r1, b1_r1, W1_r2, b1_r2,
           W2_r0, b2_r0, W2_r1, b2_r1, W2_r2, b2_r2):
    rels = [edge_index_r0, edge_index_r1, edge_index_r2]

    sidx = [_pad_idx(e[0]) for e in rels]
    didx = [_pad_idx(e[1]) for e in rels]
    sd_idx = jnp.concatenate(
        [src.astype(jnp.int32).reshape(NW, _GROWS_W // 2, K),
         dst.astype(jnp.int32).reshape(NW, _GROWS_W // 2, K)], axis=1)
    emb_pad = jnp.pad(emb, ((0, NP - N), (0, 0)))
    W1 = [W1_r0, W1_r1, W1_r2]
    W2 = [W2_r0, W2_r1, W2_r2]
    b1s = jnp.stack([b1_r0, b1_r1, b1_r2])
    b2s = jnp.stack([b2_r0, b2_r1, b2_r2])

    tab1, norms = [], []
    for r in range(3):
        degp = _sc_degrees(sidx[r], didx[r]).reshape(2, NW, NP)
        t, n = _tc_scale_emb(degp, emb_pad)
        tab1.append(t)
        norms.append(n)

    part1 = [_sc_edge_pass(tab1[r], sidx[r], didx[r]).reshape(NC, NP, D)
             for r in range(3)]
    y0 = _tc_partial_matmul(part1[0], norms[0], W1[0])
    y1 = _tc_partial_matmul(part1[1], norms[1], W1[1])
    tab2 = _tc_combine(y0, y1, part1[2], norms[0], norms[1], norms[2],
                       W1[2], b1s, True)

    part2 = [_sc_edge_pass(tab2[r], sidx[r], didx[r]).reshape(NC, NP, D)
             for r in range(3)]
    z0 = _tc_partial_matmul(part2[0], norms[0], W2[0])
    z1 = _tc_partial_matmul(part2[1], norms[1], W2[1])
    (h2,) = _tc_combine(z0, z1, part2[2], norms[0], norms[1], norms[2],
                        W2[2], b2s, False)

    return tuple(_sc_final_gather(h2, sd_idx))


# speedup vs baseline: 7.0622x; 1.0003x over previous
import dataclasses
import functools

import jax
import jax.numpy as jnp
from jax import lax
from jax.experimental import pallas as pl
from jax.experimental.pallas import tpu as pltpu
from jax.experimental.pallas import tpu_sc as plsc

N = 10000
D = 128
E = 100000
B = 8192

NP = 10240
NC, NS, NW = 2, 16, 32
K = 128
ROWS_W = 25
EPAD = NW * ROWS_W * K
RPS = NP // NS
NBUF = 2
ZROWS = 32

_mesh = plsc.VectorSubcoreMesh(core_axis_name="c", subcore_axis_name="s",
                               num_cores=NC, num_subcores=NS)

_sc_params = pltpu.CompilerParams()
if "needs_layout_passes" in pltpu.CompilerParams.__dataclass_fields__:
    _sc_params = dataclasses.replace(_sc_params, needs_layout_passes=False)


def _zfill_f32(ref, nrows, ncols16):
    @pl.loop(0, nrows)
    def _(i):
        for c in range(ncols16):
            ref[i, pl.ds(c * 16, 16)] = jnp.zeros((16,), jnp.float32)


def _deg_kernel_body(sidx_hbm, didx_hbm, out_hbm, ibuf, deg_v):
    cid = lax.axis_index("c")
    sid = lax.axis_index("s")
    wid = cid * NS + sid

    pltpu.sync_copy(sidx_hbm.at[wid], ibuf.at[pl.ds(0, ROWS_W)])
    pltpu.sync_copy(didx_hbm.at[wid], ibuf.at[pl.ds(ROWS_W, ROWS_W)])
    ones = jnp.ones((16,), jnp.float32)

    for a in range(2):
        @pl.loop(0, NP // 16)
        def _(i):
            deg_v[pl.ds(i * 16, 16)] = jnp.zeros((16,), jnp.float32)

        @pl.loop(0, ROWS_W)
        def _(j):
            for c in range(K // 16):
                idx = ibuf[a * ROWS_W + j, pl.ds(c * 16, 16)]
                plsc.addupdate_scatter(deg_v, [idx], ones)

        pltpu.sync_copy(deg_v, out_hbm.at[pl.ds((a * NW + wid) * NP, NP)])


def _sc_degrees(sidx_r, didx_r):
    f = pl.kernel(
        _deg_kernel_body,
        out_type=jax.ShapeDtypeStruct((2 * NW * NP,), jnp.float32),
        mesh=_mesh,
        scratch_types=[
            pltpu.VMEM((2 * ROWS_W, K), jnp.int32),
            pltpu.VMEM((NP,), jnp.float32),
        ],
        compiler_params=_sc_params,
    )
    return f(sidx_r, didx_r)


def _edge_kernel_body(tab_hbm, sidx_hbm, didx_hbm, out_hbm,
                      acc, sbuf, dbuf, rows, zv, gsem, ssem):
    cid = lax.axis_index("c")
    sid = lax.axis_index("s")
    wid = cid * NS + sid

    _zfill_f32(zv, ZROWS, D // 16)

    @pl.loop(0, RPS // ZROWS)
    def _(t):
        pltpu.sync_copy(zv, acc.at[pl.ds(sid * RPS + t * ZROWS, ZROWS)])
    plsc.subcore_barrier()

    pltpu.sync_copy(sidx_hbm.at[wid], sbuf)
    pltpu.sync_copy(didx_hbm.at[wid], dbuf)

    gathers = {}
    scatters = {}
    for j in range(min(NBUF, ROWS_W)):
        b = j % NBUF
        gathers[j] = pltpu.async_copy(
            tab_hbm.at[sbuf.at[j]], rows.at[pl.ds(b * K, K)], gsem.at[b])
    for j in range(ROWS_W):
        b = j % NBUF
        gathers[j].wait()
        scatters[j] = pltpu.async_copy(
            rows.at[pl.ds(b * K, K)], acc.at[dbuf.at[j]], ssem.at[b],
            add=True)
        nj = j + NBUF
        if nj < ROWS_W:
            scatters[j].wait()
            gathers[nj] = pltpu.async_copy(
                tab_hbm.at[sbuf.at[nj]], rows.at[pl.ds(b * K, K)],
                gsem.at[b])
    for j in range(max(0, ROWS_W - NBUF), ROWS_W):
        scatters[j].wait()
    plsc.subcore_barrier()

    pltpu.sync_copy(acc.at[pl.ds(sid * RPS, RPS)],
                    out_hbm.at[pl.ds(cid * NP + sid * RPS, RPS)])


def _sc_edge_pass(tab_r, sidx_r, didx_r):
    f = pl.kernel(
        _edge_kernel_body,
        out_type=jax.ShapeDtypeStruct((NC * NP, D), jnp.float32),
        mesh=_mesh,
        scratch_types=[
            pltpu.VMEM_SHARED((NP, D), jnp.float32),
            pltpu.VMEM((ROWS_W, K), jnp.int32),
            pltpu.VMEM((ROWS_W, K), jnp.int32),
            pltpu.VMEM((NBUF * K, D), jnp.float32),
            pltpu.VMEM((ZROWS, D), jnp.float32),
            pltpu.SemaphoreType.DMA((NBUF,)),
            pltpu.SemaphoreType.DMA((NBUF,)),
        ],
    )
    return f(tab_r, sidx_r, didx_r)


_GROWS_W = (2 * B) // K // NW


def _final_gather_body(h2_hbm, idx_hbm, o1_hbm, o2_hbm, ibuf, rows):
    wid = lax.axis_index("c") * NS + lax.axis_index("s")
    pltpu.sync_copy(idx_hbm.at[wid], ibuf)
    half = _GROWS_W // 2

    @pl.loop(0, half)
    def _(j):
        pltpu.sync_copy(h2_hbm.at[ibuf.at[j]], rows)
        pltpu.sync_copy(rows, o1_hbm.at[pl.ds(wid * half * K + j * K, K)])

    @pl.loop(0, half)
    def _(j):
        pltpu.sync_copy(h2_hbm.at[ibuf.at[half + j]], rows)
        pltpu.sync_copy(rows, o2_hbm.at[pl.ds(wid * half * K + j * K, K)])


def _sc_final_gather(h2, sd_idx):
    f = pl.kernel(
        _final_gather_body,
        out_type=[jax.ShapeDtypeStruct((B, D), jnp.float32),
                  jax.ShapeDtypeStruct((B, D), jnp.float32)],
        mesh=_mesh,
        scratch_types=[
            pltpu.VMEM((_GROWS_W, K), jnp.int32),
            pltpu.VMEM((K, D), jnp.float32),
        ],
    )
    return f(h2, sd_idx)


BLK = 1024


def _tca_body(degp_ref, emb_ref, tab_ref, norms_ref):
    deg = jnp.sum(degp_ref[...], axis=1)
    norms = lax.rsqrt(jnp.maximum(deg, 1.0))
    norms_ref[...] = norms
    tab_ref[...] = emb_ref[...] * norms[0][:, None]


def _tc_scale_emb(degp_r, emb_pad):
    return pl.pallas_call(
        _tca_body,
        grid=(NP // BLK,),
        in_specs=[
            pl.BlockSpec((2, NW, BLK), lambda i: (0, 0, i)),
            pl.BlockSpec((BLK, D), lambda i: (i, 0)),
        ],
        out_specs=[
            pl.BlockSpec((BLK, D), lambda i: (i, 0)),
            pl.BlockSpec((2, BLK), lambda i: (0, i)),
        ],
        out_shape=[
            jax.ShapeDtypeStruct((NP, D), jnp.float32),
            jax.ShapeDtypeStruct((2, NP), jnp.float32),
        ],
    )(degp_r, emb_pad)


def _tcy_body(p0_ref, n0_ref, w0_ref, p1_ref, n1_ref, w1_ref, y_ref):
    x0 = (p0_ref[0] + p0_ref[1]) * n0_ref[1][:, None]
    x1 = (p1_ref[0] + p1_ref[1]) * n1_ref[1][:, None]
    y_ref[...] = (
        jnp.dot(x0, w0_ref[...], preferred_element_type=jnp.float32)
        + jnp.dot(x1, w1_ref[...], preferred_element_type=jnp.float32))


def _tc_partial_matmul(p0, n0, W0, p1, n1, W1):
    pblk = pl.BlockSpec((2, BLK, D), lambda i: (0, i, 0))
    nblk = pl.BlockSpec((2, BLK), lambda i: (0, i))
    wblk = pl.BlockSpec((D, D), lambda i: (0, 0))
    return pl.pallas_call(
        _tcy_body,
        grid=(NP // BLK,),
        in_specs=[pblk, nblk, wblk, pblk, nblk, wblk],
        out_specs=pl.BlockSpec((BLK, D), lambda i: (i, 0)),
        out_shape=jax.ShapeDtypeStruct((NP, D), jnp.float32),
    )(p0, n0, W0, p1, n1, W1)


def _tcf_tables_body(ys_ref, part_ref, n0_ref, n1_ref, n2_ref,
                     w_ref, b_ref, o0_ref, o1_ref, o2_ref):
    bsum = jnp.sum(b_ref[...], axis=0)
    x = (part_ref[0] + part_ref[1]) * n2_ref[1][:, None]
    acc = (ys_ref[...] + bsum[None, :]
           + jnp.dot(x, w_ref[...], preferred_element_type=jnp.float32))
    h = jnp.tanh(acc)
    o0_ref[...] = h * n0_ref[0][:, None]
    o1_ref[...] = h * n1_ref[0][:, None]
    o2_ref[...] = h * n2_ref[0][:, None]


def _tcf_final_body(ys_ref, part_ref, n0_ref, n1_ref, n2_ref,
                    w_ref, b_ref, o0_ref):
    bsum = jnp.sum(b_ref[...], axis=0)
    x = (part_ref[0] + part_ref[1]) * n2_ref[1][:, None]
    acc = (ys_ref[...] + bsum[None, :]
           + jnp.dot(x, w_ref[...], preferred_element_type=jnp.float32))
    o0_ref[...] = jnp.tanh(acc)


def _tc_combine(ys, part2, n0, n1, n2, W_2, bs, make_tables):
    nblk = pl.BlockSpec((2, BLK), lambda i: (0, i))
    row = pl.BlockSpec((BLK, D), lambda i: (i, 0))
    if make_tables:
        body = _tcf_tables_body
        out_specs = [row, row, row]
        out_shape = [jax.ShapeDtypeStruct((NP, D), jnp.float32)] * 3
    else:
        body = _tcf_final_body
        out_specs = [row]
        out_shape = [jax.ShapeDtypeStruct((NP, D), jnp.float32)]
    return pl.pallas_call(
        body,
        grid=(NP // BLK,),
        in_specs=[
            row,
            pl.BlockSpec((2, BLK, D), lambda i: (0, i, 0)),
            nblk, nblk, nblk,
            pl.BlockSpec((D, D), lambda i: (0, 0)),
            pl.BlockSpec((3, D), lambda i: (0, 0)),
        ],
        out_specs=out_specs,
        out_shape=out_shape,
    )(ys, part2, n0, n1, n2, W_2, bs)


def _pad_idx(a):
    a = a.astype(jnp.int32)
    pad = N + (jnp.arange(EPAD - E, dtype=jnp.int32) % (NP - N))
    return jnp.concatenate([a, pad]).reshape(NW, ROWS_W, K)


def kernel(edge_index_r0, edge_index_r1, edge_index_r2, src, dst, emb,
           W1_r0, b1_r0, W1_r1, b1_r1, W1_r2, b1_r2,
           W2_r0, b2_r0, W2_r1, b2_r1, W2_r2, b2_r2):
    rels = [edge_index_r0, edge_index_r1, edge_index_r2]

    sidx = [_pad_idx(e[0]) for e in rels]
    didx = [_pad_idx(e[1]) for e in rels]
    sd_idx = jnp.concatenate(
        [src.astype(jnp.int32).reshape(NW, _GROWS_W // 2, K),
         dst.astype(jnp.int32).reshape(NW, _GROWS_W // 2, K)], axis=1)
    emb_pad = jnp.pad(emb, ((0, NP - N), (0, 0)))
    W1 = [W1_r0, W1_r1, W1_r2]
    W2 = [W2_r0, W2_r1, W2_r2]
    b1s = jnp.stack([b1_r0, b1_r1, b1_r2])
    b2s = jnp.stack([b2_r0, b2_r1, b2_r2])

    tab1, norms = [], []
    for r in range(3):
        degp = _sc_degrees(sidx[r], didx[r]).reshape(2, NW, NP)
        t, n = _tc_scale_emb(degp, emb_pad)
        tab1.append(t)
        norms.append(n)

    part1 = [_sc_edge_pass(tab1[r], sidx[r], didx[r]).reshape(NC, NP, D)
             for r in range(3)]
    ys1 = _tc_partial_matmul(part1[0], norms[0], W1[0],
                             part1[1], norms[1], W1[1])
    tab2 = _tc_combine(ys1, part1[2], norms[0], norms[1], norms[2],
                       W1[2], b1s, True)

    part2 = [_sc_edge_pass(tab2[r], sidx[r], didx[r]).reshape(NC, NP, D)
             for r in range(3)]
    ys2 = _tc_partial_matmul(part2[0], norms[0], W2[0],
                             part2[1], norms[1], W2[1])
    (h2,) = _tc_combine(ys2, part2[2], norms[0], norms[1], norms[2],
                        W2[2], b2s, False)

    return tuple(_sc_final_gather(h2, sd_idx))


# speedup vs baseline: 7.0638x; 1.0002x over previous
import dataclasses
import functools

import jax
import jax.numpy as jnp
from jax import lax
from jax.experimental import pallas as pl
from jax.experimental.pallas import tpu as pltpu
from jax.experimental.pallas import tpu_sc as plsc

N = 10000
D = 128
E = 100000
B = 8192

NP = 10240
NC, NS, NW = 2, 16, 32
K = 128
ROWS_W = 25
EPAD = NW * ROWS_W * K
RPS = NP // NS
NBUF = 2
ZROWS = 32

_mesh = plsc.VectorSubcoreMesh(core_axis_name="c", subcore_axis_name="s",
                               num_cores=NC, num_subcores=NS)

_sc_params = pltpu.CompilerParams()
if "needs_layout_passes" in pltpu.CompilerParams.__dataclass_fields__:
    _sc_params = dataclasses.replace(_sc_params, needs_layout_passes=False)


def _zfill_f32(ref, nrows, ncols16):
    @pl.loop(0, nrows)
    def _(i):
        for c in range(ncols16):
            ref[i, pl.ds(c * 16, 16)] = jnp.zeros((16,), jnp.float32)


def _deg_kernel_body(sidx_hbm, didx_hbm, out_hbm, ibuf, deg_v):
    cid = lax.axis_index("c")
    sid = lax.axis_index("s")
    wid = cid * NS + sid

    pltpu.sync_copy(sidx_hbm.at[wid], ibuf.at[pl.ds(0, ROWS_W)])
    pltpu.sync_copy(didx_hbm.at[wid], ibuf.at[pl.ds(ROWS_W, ROWS_W)])
    ones = jnp.ones((16,), jnp.float32)

    for a in range(2):
        @pl.loop(0, NP // 16)
        def _(i):
            deg_v[pl.ds(i * 16, 16)] = jnp.zeros((16,), jnp.float32)

        @pl.loop(0, ROWS_W)
        def _(j):
            for c in range(K // 16):
                idx = ibuf[a * ROWS_W + j, pl.ds(c * 16, 16)]
                plsc.addupdate_scatter(deg_v, [idx], ones)

        pltpu.sync_copy(deg_v, out_hbm.at[pl.ds((a * NW + wid) * NP, NP)])


def _sc_degrees(sidx_r, didx_r):
    f = pl.kernel(
        _deg_kernel_body,
        out_type=jax.ShapeDtypeStruct((2 * NW * NP,), jnp.float32),
        mesh=_mesh,
        scratch_types=[
            pltpu.VMEM((2 * ROWS_W, K), jnp.int32),
            pltpu.VMEM((NP,), jnp.float32),
        ],
        compiler_params=_sc_params,
    )
    return f(sidx_r, didx_r)


def _edge_kernel_body(tab_hbm, sidx_hbm, didx_hbm, out_hbm,
                      acc, sbuf, dbuf, rows, zv, gsem, ssem):
    cid = lax.axis_index("c")
    sid = lax.axis_index("s")
    wid = cid * NS + sid

    _zfill_f32(zv, ZROWS, D // 16)

    @pl.loop(0, RPS // ZROWS)
    def _(t):
        pltpu.sync_copy(zv, acc.at[pl.ds(sid * RPS + t * ZROWS, ZROWS)])
    plsc.subcore_barrier()

    pltpu.sync_copy(sidx_hbm.at[wid], sbuf)
    pltpu.sync_copy(didx_hbm.at[wid], dbuf)

    gathers = {}
    scatters = {}
    for j in range(min(NBUF, ROWS_W)):
        b = j % NBUF
        gathers[j] = pltpu.async_copy(
            tab_hbm.at[sbuf.at[j]], rows.at[pl.ds(b * K, K)], gsem.at[b])
    for j in range(ROWS_W):
        b = j % NBUF
        gathers[j].wait()
        scatters[j] = pltpu.async_copy(
            rows.at[pl.ds(b * K, K)], acc.at[dbuf.at[j]], ssem.at[b],
            add=True)
        nj = j + NBUF
        if nj < ROWS_W:
            scatters[j].wait()
            gathers[nj] = pltpu.async_copy(
                tab_hbm.at[sbuf.at[nj]], rows.at[pl.ds(b * K, K)],
                gsem.at[b])
    for j in range(max(0, ROWS_W - NBUF), ROWS_W):
        scatters[j].wait()
    plsc.subcore_barrier()

    pltpu.sync_copy(acc.at[pl.ds(sid * RPS, RPS)],
                    out_hbm.at[pl.ds(cid * NP + sid * RPS, RPS)])


def _sc_edge_pass(tab_r, sidx_r, didx_r):
    f = pl.kernel(
        _edge_kernel_body,
        out_type=jax.ShapeDtypeStruct((NC * NP, D), jnp.float32),
        mesh=_mesh,
        scratch_types=[
            pltpu.VMEM_SHARED((NP, D), jnp.float32),
            pltpu.VMEM((ROWS_W, K), jnp.int32),
            pltpu.VMEM((ROWS_W, K), jnp.int32),
            pltpu.VMEM((NBUF * K, D), jnp.float32),
            pltpu.VMEM((ZROWS, D), jnp.float32),
            pltpu.SemaphoreType.DMA((NBUF,)),
            pltpu.SemaphoreType.DMA((NBUF,)),
        ],
    )
    return f(tab_r, sidx_r, didx_r)


_GROWS_W = (2 * B) // K // NW


def _final_gather_body(h2_hbm, idx_hbm, o1_hbm, o2_hbm, ibuf, rows):
    wid = lax.axis_index("c") * NS + lax.axis_index("s")
    pltpu.sync_copy(idx_hbm.at[wid], ibuf)
    half = _GROWS_W // 2

    @pl.loop(0, half)
    def _(j):
        pltpu.sync_copy(h2_hbm.at[ibuf.at[j]], rows)
        pltpu.sync_copy(rows, o1_hbm.at[pl.ds(wid * half * K + j * K, K)])

    @pl.loop(0, half)
    def _(j):
        pltpu.sync_copy(h2_hbm.at[ibuf.at[half + j]], rows)
        pltpu.sync_copy(rows, o2_hbm.at[pl.ds(wid * half * K + j * K, K)])


def _sc_final_gather(h2, sd_idx):
    f = pl.kernel(
        _final_gather_body,
        out_type=[jax.ShapeDtypeStruct((B, D), jnp.float32),
                  jax.ShapeDtypeStruct((B, D), jnp.float32)],
        mesh=_mesh,
        scratch_types=[
            pltpu.VMEM((_GROWS_W, K), jnp.int32),
            pltpu.VMEM((K, D), jnp.float32),
        ],
    )
    return f(h2, sd_idx)


BLK = 1024


def _tca_body(degp_ref, emb_ref, tab_ref, norms_ref):
    deg = jnp.sum(degp_ref[...], axis=1)
    norms = lax.rsqrt(jnp.maximum(deg, 1.0))
    norms_ref[...] = norms
    tab_ref[...] = emb_ref[...] * norms[0][:, None]


def _tc_scale_emb(degp_r, emb_pad):
    return pl.pallas_call(
        _tca_body,
        grid=(NP // BLK,),
        in_specs=[
            pl.BlockSpec((2, NW, BLK), lambda i: (0, 0, i)),
            pl.BlockSpec((BLK, D), lambda i: (i, 0)),
        ],
        out_specs=[
            pl.BlockSpec((BLK, D), lambda i: (i, 0)),
            pl.BlockSpec((2, BLK), lambda i: (0, i)),
        ],
        out_shape=[
            jax.ShapeDtypeStruct((NP, D), jnp.float32),
            jax.ShapeDtypeStruct((2, NP), jnp.float32),
        ],
    )(degp_r, emb_pad)


def _tcy_body(p0_ref, n0_ref, w0_ref, p1_ref, n1_ref, w1_ref, y_ref):
    x0 = (p0_ref[0] + p0_ref[1]) * n0_ref[1][:, None]
    x1 = (p1_ref[0] + p1_ref[1]) * n1_ref[1][:, None]
    y_ref[...] = (
        jnp.dot(x0, w0_ref[...], preferred_element_type=jnp.float32)
        + jnp.dot(x1, w1_ref[...], preferred_element_type=jnp.float32))


def _tc_partial_matmul(p0, n0, W0, p1, n1, W1):
    pblk = pl.BlockSpec((2, BLK, D), lambda i: (0, i, 0))
    nblk = pl.BlockSpec((2, BLK), lambda i: (0, i))
    wblk = pl.BlockSpec((D, D), lambda i: (0, 0))
    return pl.pallas_call(
        _tcy_body,
        grid=(NP // BLK,),
        in_specs=[pblk, nblk, wblk, pblk, nblk, wblk],
        out_specs=pl.BlockSpec((BLK, D), lambda i: (i, 0)),
        out_shape=jax.ShapeDtypeStruct((NP, D), jnp.float32),
    )(p0, n0, W0, p1, n1, W1)


def _tcf_tab0_body(ys_ref, part_ref, n0_ref, n1_ref, n2_ref,
                   w_ref, b_ref, o0_ref):
    bsum = jnp.sum(b_ref[...], axis=0)
    x = (part_ref[0] + part_ref[1]) * n2_ref[1][:, None]
    acc = (ys_ref[...] + bsum[None, :]
           + jnp.dot(x, w_ref[...], preferred_element_type=jnp.float32))
    h = jnp.tanh(acc)
    o0_ref[...] = h * n0_ref[0][:, None]


def _tcf_tab12_body(ys_ref, part_ref, n0_ref, n1_ref, n2_ref,
                    w_ref, b_ref, o1_ref, o2_ref):
    bsum = jnp.sum(b_ref[...], axis=0)
    x = (part_ref[0] + part_ref[1]) * n2_ref[1][:, None]
    acc = (ys_ref[...] + bsum[None, :]
           + jnp.dot(x, w_ref[...], preferred_element_type=jnp.float32))
    h = jnp.tanh(acc)
    o1_ref[...] = h * n1_ref[0][:, None]
    o2_ref[...] = h * n2_ref[0][:, None]


def _tcf_final_body(ys_ref, part_ref, n0_ref, n1_ref, n2_ref,
                    w_ref, b_ref, o0_ref):
    bsum = jnp.sum(b_ref[...], axis=0)
    x = (part_ref[0] + part_ref[1]) * n2_ref[1][:, None]
    acc = (ys_ref[...] + bsum[None, :]
           + jnp.dot(x, w_ref[...], preferred_element_type=jnp.float32))
    o0_ref[...] = jnp.tanh(acc)


def _tc_combine(ys, part2, n0, n1, n2, W_2, bs, make_tables):
    nblk = pl.BlockSpec((2, BLK), lambda i: (0, i))
    row = pl.BlockSpec((BLK, D), lambda i: (i, 0))
    if make_tables == "tab0":
        body = _tcf_tab0_body
        out_specs = [row]
        out_shape = [jax.ShapeDtypeStruct((NP, D), jnp.float32)]
    elif make_tables == "tab12":
        body = _tcf_tab12_body
        out_specs = [row, row]
        out_shape = [jax.ShapeDtypeStruct((NP, D), jnp.float32)] * 2
    else:
        body = _tcf_final_body
        out_specs = [row]
        out_shape = [jax.ShapeDtypeStruct((NP, D), jnp.float32)]
    return pl.pallas_call(
        body,
        grid=(NP // BLK,),
        in_specs=[
            row,
            pl.BlockSpec((2, BLK, D), lambda i: (0, i, 0)),
            nblk, nblk, nblk,
            pl.BlockSpec((D, D), lambda i: (0, 0)),
            pl.BlockSpec((3, D), lambda i: (0, 0)),
        ],
        out_specs=out_specs,
        out_shape=out_shape,
    )(ys, part2, n0, n1, n2, W_2, bs)


def _pad_idx(a):
    a = a.astype(jnp.int32)
    pad = N + (jnp.arange(EPAD - E, dtype=jnp.int32) % (NP - N))
    return jnp.concatenate([a, pad]).reshape(NW, ROWS_W, K)


def kernel(edge_index_r0, edge_index_r1, edge_index_r2, src, dst, emb,
           W1_r0, b1_r0, W1_r1, b1_r1, W1_r2, b1_r2,
           W2_r0, b2_r0, W2_r1, b2_r1, W2_r2, b2_r2):
    rels = [edge_index_r0, edge_index_r1, edge_index_r2]

    sidx = [_pad_idx(e[0]) for e in rels]
    didx = [_pad_idx(e[1]) for e in rels]
    sd_idx = jnp.concatenate(
        [src.astype(jnp.int32).reshape(NW, _GROWS_W // 2, K),
         dst.astype(jnp.int32).reshape(NW, _GROWS_W // 2, K)], axis=1)
    emb_pad = jnp.pad(emb, ((0, NP - N), (0, 0)))
    W1 = [W1_r0, W1_r1, W1_r2]
    W2 = [W2_r0, W2_r1, W2_r2]
    b1s = jnp.stack([b1_r0, b1_r1, b1_r2])
    b2s = jnp.stack([b2_r0, b2_r1, b2_r2])

    tab1, norms = [], []
    for r in range(3):
        degp = _sc_degrees(sidx[r], didx[r]).reshape(2, NW, NP)
        t, n = _tc_scale_emb(degp, emb_pad)
        tab1.append(t)
        norms.append(n)

    part1 = [_sc_edge_pass(tab1[r], sidx[r], didx[r]).reshape(NC, NP, D)
             for r in range(3)]
    ys1 = _tc_partial_matmul(part1[0], norms[0], W1[0],
                             part1[1], norms[1], W1[1])
    (tab2_0,) = _tc_combine(ys1, part1[2], norms[0], norms[1], norms[2],
                            W1[2], b1s, "tab0")
    tab2_12 = _tc_combine(ys1, part1[2], norms[0], norms[1], norms[2],
                          W1[2], b1s, "tab12")
    tab2 = [tab2_0] + list(tab2_12)

    part2 = [_sc_edge_pass(tab2[r], sidx[r], didx[r]).reshape(NC, NP, D)
             for r in range(3)]
    ys2 = _tc_partial_matmul(part2[0], norms[0], W2[0],
                             part2[1], norms[1], W2[1])
    (h2,) = _tc_combine(ys2, part2[2], norms[0], norms[1], norms[2],
                        W2[2], b2s, False)

    return tuple(_sc_final_gather(h2, sd_idx))


# speedup vs baseline: 7.1435x; 1.0113x over previous
import dataclasses
import functools

import jax
import jax.numpy as jnp
from jax import lax
from jax.experimental import pallas as pl
from jax.experimental.pallas import tpu as pltpu
from jax.experimental.pallas import tpu_sc as plsc

N = 10000
D = 128
E = 100000
B = 8192

NP = 10240
NC, NS, NW = 2, 16, 32
K = 128
ROWS_W = 25
EPAD = NW * ROWS_W * K
RPS = NP // NS
NBUF = 2
ZROWS = 32

_mesh = plsc.VectorSubcoreMesh(core_axis_name="c", subcore_axis_name="s",
                               num_cores=NC, num_subcores=NS)

_sc_params = pltpu.CompilerParams()
if "needs_layout_passes" in pltpu.CompilerParams.__dataclass_fields__:
    _sc_params = dataclasses.replace(_sc_params, needs_layout_passes=False)


def _zfill_f32(ref, nrows, ncols16):
    @pl.loop(0, nrows)
    def _(i):
        for c in range(ncols16):
            ref[i, pl.ds(c * 16, 16)] = jnp.zeros((16,), jnp.float32)


def _deg_kernel_body(sidx_hbm, didx_hbm, out_hbm, ibuf, deg_v):
    cid = lax.axis_index("c")
    sid = lax.axis_index("s")
    wid = cid * NS + sid

    pltpu.sync_copy(sidx_hbm.at[wid], ibuf.at[pl.ds(0, ROWS_W)])
    pltpu.sync_copy(didx_hbm.at[wid], ibuf.at[pl.ds(ROWS_W, ROWS_W)])
    ones = jnp.ones((16,), jnp.float32)

    for a in range(2):
        @pl.loop(0, NP // 16)
        def _(i):
            deg_v[pl.ds(i * 16, 16)] = jnp.zeros((16,), jnp.float32)

        @pl.loop(0, ROWS_W)
        def _(j):
            for c in range(K // 16):
                idx = ibuf[a * ROWS_W + j, pl.ds(c * 16, 16)]
                plsc.addupdate_scatter(deg_v, [idx], ones)

        pltpu.sync_copy(deg_v, out_hbm.at[pl.ds((a * NW + wid) * NP, NP)])


def _sc_degrees(sidx_r, didx_r):
    f = pl.kernel(
        _deg_kernel_body,
        out_type=jax.ShapeDtypeStruct((2 * NW * NP,), jnp.float32),
        mesh=_mesh,
        scratch_types=[
            pltpu.VMEM((2 * ROWS_W, K), jnp.int32),
            pltpu.VMEM((NP,), jnp.float32),
        ],
        compiler_params=_sc_params,
    )
    return f(sidx_r, didx_r)


def _edge_kernel_body(tab_hbm, sidx_hbm, didx_hbm, out_hbm,
                      acc, sbuf, dbuf, rows, zv, gsem, ssem):
    cid = lax.axis_index("c")
    sid = lax.axis_index("s")
    wid = cid * NS + sid

    _zfill_f32(zv, ZROWS, D // 16)

    zeros = [pltpu.async_copy(
                 zv, acc.at[pl.ds(sid * RPS + t * ZROWS, ZROWS)], gsem.at[0])
             for t in range(RPS // ZROWS)]
    for z in zeros:
        z.wait()
    plsc.subcore_barrier()

    pltpu.sync_copy(sidx_hbm.at[wid], sbuf)
    pltpu.sync_copy(didx_hbm.at[wid], dbuf)

    gathers = {}
    scatters = {}
    for j in range(min(NBUF, ROWS_W)):
        b = j % NBUF
        gathers[j] = pltpu.async_copy(
            tab_hbm.at[sbuf.at[j]], rows.at[pl.ds(b * K, K)], gsem.at[b])
    for j in range(ROWS_W):
        b = j % NBUF
        gathers[j].wait()
        scatters[j] = pltpu.async_copy(
            rows.at[pl.ds(b * K, K)], acc.at[dbuf.at[j]], ssem.at[b],
            add=True)
        nj = j + NBUF
        if nj < ROWS_W:
            scatters[j].wait()
            gathers[nj] = pltpu.async_copy(
                tab_hbm.at[sbuf.at[nj]], rows.at[pl.ds(b * K, K)],
                gsem.at[b])
    for j in range(max(0, ROWS_W - NBUF), ROWS_W):
        scatters[j].wait()
    plsc.subcore_barrier()

    pltpu.sync_copy(acc.at[pl.ds(sid * RPS, RPS)],
                    out_hbm.at[pl.ds(cid * NP + sid * RPS, RPS)])


def _sc_edge_pass(tab_r, sidx_r, didx_r):
    f = pl.kernel(
        _edge_kernel_body,
        out_type=jax.ShapeDtypeStruct((NC * NP, D), jnp.float32),
        mesh=_mesh,
        scratch_types=[
            pltpu.VMEM_SHARED((NP, D), jnp.float32),
            pltpu.VMEM((ROWS_W, K), jnp.int32),
            pltpu.VMEM((ROWS_W, K), jnp.int32),
            pltpu.VMEM((NBUF * K, D), jnp.float32),
            pltpu.VMEM((ZROWS, D), jnp.float32),
            pltpu.SemaphoreType.DMA((NBUF,)),
            pltpu.SemaphoreType.DMA((NBUF,)),
        ],
    )
    return f(tab_r, sidx_r, didx_r)


_GROWS_W = (2 * B) // K // NW


def _final_gather_body(h2_hbm, idx_hbm, o1_hbm, o2_hbm, ibuf, rows):
    wid = lax.axis_index("c") * NS + lax.axis_index("s")
    pltpu.sync_copy(idx_hbm.at[wid], ibuf)
    half = _GROWS_W // 2

    @pl.loop(0, half)
    def _(j):
        pltpu.sync_copy(h2_hbm.at[ibuf.at[j]], rows)
        pltpu.sync_copy(rows, o1_hbm.at[pl.ds(wid * half * K + j * K, K)])

    @pl.loop(0, half)
    def _(j):
        pltpu.sync_copy(h2_hbm.at[ibuf.at[half + j]], rows)
        pltpu.sync_copy(rows, o2_hbm.at[pl.ds(wid * half * K + j * K, K)])


def _sc_final_gather(h2, sd_idx):
    f = pl.kernel(
        _final_gather_body,
        out_type=[jax.ShapeDtypeStruct((B, D), jnp.float32),
                  jax.ShapeDtypeStruct((B, D), jnp.float32)],
        mesh=_mesh,
        scratch_types=[
            pltpu.VMEM((_GROWS_W, K), jnp.int32),
            pltpu.VMEM((K, D), jnp.float32),
        ],
    )
    return f(h2, sd_idx)


BLK = 1024


def _tca_body(degp_ref, emb_ref, tab_ref, norms_ref):
    deg = jnp.sum(degp_ref[...], axis=1)
    norms = lax.rsqrt(jnp.maximum(deg, 1.0))
    norms_ref[...] = norms
    tab_ref[...] = emb_ref[...] * norms[0][:, None]


def _tc_scale_emb(degp_r, emb_pad):
    return pl.pallas_call(
        _tca_body,
        grid=(NP // BLK,),
        in_specs=[
            pl.BlockSpec((2, NW, BLK), lambda i: (0, 0, i)),
            pl.BlockSpec((BLK, D), lambda i: (i, 0)),
        ],
        out_specs=[
            pl.BlockSpec((BLK, D), lambda i: (i, 0)),
            pl.BlockSpec((2, BLK), lambda i: (0, i)),
        ],
        out_shape=[
            jax.ShapeDtypeStruct((NP, D), jnp.float32),
            jax.ShapeDtypeStruct((2, NP), jnp.float32),
        ],
    )(degp_r, emb_pad)


def _tcy_body(p0_ref, n0_ref, w0_ref, p1_ref, n1_ref, w1_ref, y_ref):
    x0 = (p0_ref[0] + p0_ref[1]) * n0_ref[1][:, None]
    x1 = (p1_ref[0] + p1_ref[1]) * n1_ref[1][:, None]
    y_ref[...] = (
        jnp.dot(x0, w0_ref[...], preferred_element_type=jnp.float32)
        + jnp.dot(x1, w1_ref[...], preferred_element_type=jnp.float32))


def _tc_partial_matmul(p0, n0, W0, p1, n1, W1):
    pblk = pl.BlockSpec((2, BLK, D), lambda i: (0, i, 0))
    nblk = pl.BlockSpec((2, BLK), lambda i: (0, i))
    wblk = pl.BlockSpec((D, D), lambda i: (0, 0))
    return pl.pallas_call(
        _tcy_body,
        grid=(NP // BLK,),
        in_specs=[pblk, nblk, wblk, pblk, nblk, wblk],
        out_specs=pl.BlockSpec((BLK, D), lambda i: (i, 0)),
        out_shape=jax.ShapeDtypeStruct((NP, D), jnp.float32),
    )(p0, n0, W0, p1, n1, W1)


def _tcf_tab0_body(ys_ref, part_ref, n0_ref, n1_ref, n2_ref,
                   w_ref, b_ref, o0_ref):
    bsum = jnp.sum(b_ref[...], axis=0)
    x = (part_ref[0] + part_ref[1]) * n2_ref[1][:, None]
    acc = (ys_ref[...] + bsum[None, :]
           + jnp.dot(x, w_ref[...], preferred_element_type=jnp.float32))
    h = jnp.tanh(acc)
    o0_ref[...] = h * n0_ref[0][:, None]


def _tcf_tab12_body(ys_ref, part_ref, n0_ref, n1_ref, n2_ref,
                    w_ref, b_ref, o1_ref, o2_ref):
    bsum = jnp.sum(b_ref[...], axis=0)
    x = (part_ref[0] + part_ref[1]) * n2_ref[1][:, None]
    acc = (ys_ref[...] + bsum[None, :]
           + jnp.dot(x, w_ref[...], preferred_element_type=jnp.float32))
    h = jnp.tanh(acc)
    o1_ref[...] = h * n1_ref[0][:, None]
    o2_ref[...] = h * n2_ref[0][:, None]


def _tcf_final_body(ys_ref, part_ref, n0_ref, n1_ref, n2_ref,
                    w_ref, b_ref, o0_ref):
    bsum = jnp.sum(b_ref[...], axis=0)
    x = (part_ref[0] + part_ref[1]) * n2_ref[1][:, None]
    acc = (ys_ref[...] + bsum[None, :]
           + jnp.dot(x, w_ref[...], preferred_element_type=jnp.float32))
    o0_ref[...] = jnp.tanh(acc)


def _tc_combine(ys, part2, n0, n1, n2, W_2, bs, make_tables):
    nblk = pl.BlockSpec((2, BLK), lambda i: (0, i))
    row = pl.BlockSpec((BLK, D), lambda i: (i, 0))
    if make_tables == "tab0":
        body = _tcf_tab0_body
        out_specs = [row]
        out_shape = [jax.ShapeDtypeStruct((NP, D), jnp.float32)]
    elif make_tables == "tab12":
        body = _tcf_tab12_body
        out_specs = [row, row]
        out_shape = [jax.ShapeDtypeStruct((NP, D), jnp.float32)] * 2
    else:
        body = _tcf_final_body
        out_specs = [row]
        out_shape = [jax.ShapeDtypeStruct((NP, D), jnp.float32)]
    return pl.pallas_call(
        body,
        grid=(NP // BLK,),
        in_specs=[
            row,
            pl.BlockSpec((2, BLK, D), lambda i: (0, i, 0)),
            nblk, nblk, nblk,
            pl.BlockSpec((D, D), lambda i: (0, 0)),
            pl.BlockSpec((3, D), lambda i: (0, 0)),
        ],
        out_specs=out_specs,
        out_shape=out_shape,
    )(ys, part2, n0, n1, n2, W_2, bs)


def _pad_idx(a):
    a = a.astype(jnp.int32)
    pad = N + (jnp.arange(EPAD - E, dtype=jnp.int32) % (NP - N))
    return jnp.concatenate([a, pad]).reshape(NW, ROWS_W, K)


def kernel(edge_index_r0, edge_index_r1, edge_index_r2, src, dst, emb,
           W1_r0, b1_r0, W1_r1, b1_r1, W1_r2, b1_r2,
           W2_r0, b2_r0, W2_r1, b2_r1, W2_r2, b2_r2):
    rels = [edge_index_r0, edge_index_r1, edge_index_r2]

    sidx = [_pad_idx(e[0]) for e in rels]
    didx = [_pad_idx(e[1]) for e in rels]
    sd_idx = jnp.concatenate(
        [src.astype(jnp.int32).reshape(NW, _GROWS_W // 2, K),
         dst.astype(jnp.int32).reshape(NW, _GROWS_W // 2, K)], axis=1)
    emb_pad = jnp.pad(emb, ((0, NP - N), (0, 0)))
    W1 = [W1_r0, W1_r1, W1_r2]
    W2 = [W2_r0, W2_r1, W2_r2]
    b1s = jnp.stack([b1_r0, b1_r1, b1_r2])
    b2s = jnp.stack([b2_r0, b2_r1, b2_r2])

    tab1, norms = [], []
    for r in range(3):
        degp = _sc_degrees(sidx[r], didx[r]).reshape(2, NW, NP)
        t, n = _tc_scale_emb(degp, emb_pad)
        tab1.append(t)
        norms.append(n)

    part1 = [_sc_edge_pass(tab1[r], sidx[r], didx[r]).reshape(NC, NP, D)
             for r in range(3)]
    ys1 = _tc_partial_matmul(part1[0], norms[0], W1[0],
                             part1[1], norms[1], W1[1])
    (tab2_0,) = _tc_combine(ys1, part1[2], norms[0], norms[1], norms[2],
                            W1[2], b1s, "tab0")
    tab2_12 = _tc_combine(ys1, part1[2], norms[0], norms[1], norms[2],
                          W1[2], b1s, "tab12")
    tab2 = [tab2_0] + list(tab2_12)

    part2 = [_sc_edge_pass(tab2[r], sidx[r], didx[r]).reshape(NC, NP, D)
             for r in range(3)]
    ys2 = _tc_partial_matmul(part2[0], norms[0], W2[0],
                             part2[1], norms[1], W2[1])
    (h2,) = _tc_combine(ys2, part2[2], norms[0], norms[1], norms[2],
                        W2[2], b2s, False)

    return tuple(_sc_final_gather(h2, sd_idx))
